# Initial kernel scaffold; baseline (speedup 1.0000x reference)
import jax
import jax.numpy as jnp
from jax.experimental import pallas as pl


def kernel(x, edge_index, batch, obs, W1, b1, W2, b2, W3, b3, LW1, Lb1, LW2, Lb2, LW3, Lb3, BW1, Bb1, BW2, Bb2):
    raise NotImplementedError("write your pallas kernel here")


# speedup vs baseline: 7.5100x; 7.5100x over previous
import functools

import jax
import jax.numpy as jnp
from jax import lax
from jax.experimental import pallas as pl
from jax.experimental.pallas import tpu as pltpu
from jax.experimental.pallas import tpu_sc as plsc

_N = 10000
_D = 256
_H = 256
_G = 32
_L = 50
_E = 160000

_CHUNK = 128
_EPAD = 163840
_NCHUNK = 80
_TILE_E = _NCHUNK * _CHUNK
_ACC_ROWS = 10016
_DEG_TILE_E = _EPAD // 32
_WT = 10
_WROWS = _N // _WT
_WCH = 200

_sc_mesh = plsc.VectorSubcoreMesh(core_axis_name="c", subcore_axis_name="s",
                                  num_cores=2, num_subcores=16)


def _deg_body(dst_hbm, z8_hbm, ones_hbm, p0_hbm, p1_hbm,
                dst_v, ones_v, stage_v, acc):
    cid = lax.axis_index("c")
    sid = lax.axis_index("s")
    wid = cid * 16 + sid
    pltpu.sync_copy(ones_hbm, ones_v)
    pltpu.sync_copy(z8_hbm, acc.at[pl.ds(sid * 640, 640)])
    plsc.subcore_barrier()

    base = wid * _DEG_TILE_E

    def chunk(j, _):
        pltpu.sync_copy(dst_hbm.at[pl.ds(base + j * _CHUNK, _CHUNK)], dst_v)
        pltpu.sync_copy(ones_v, acc.at[dst_v], add=True)
        return 0

    lax.fori_loop(0, _DEG_TILE_E // _CHUNK, chunk, 0)
    plsc.subcore_barrier()

    @pl.when(sid < _WT)
    def _():
        def wout(j, _):
            r = sid * _WROWS + j * _WCH
            pltpu.sync_copy(acc.at[pl.ds(r, _WCH)], stage_v)

            @pl.when(cid == 0)
            def _():
                pltpu.sync_copy(stage_v, p0_hbm.at[pl.ds(r, _WCH)])

            @pl.when(cid == 1)
            def _():
                pltpu.sync_copy(stage_v, p1_hbm.at[pl.ds(r, _WCH)])

            return 0

        lax.fori_loop(0, _WROWS // _WCH, wout, 0)


def _edge_sum_body(src_hbm, dst_hbm, hw0_hbm, hw1_hbm, z_hbm,
                   s0_hbm, s1_hbm,
                   si0, si1, di0, di1, r0, r1, acc,
                   is0, is1, g0, g1):
    cid = lax.axis_index("c")
    sid = lax.axis_index("s")
    si = [si0, si1]
    di = [di0, di1]
    rows = [r0, r1]
    isem = [is0, is1]
    gsem = [g0, g1]

    @pl.when(sid < 15)
    def _():
        pltpu.sync_copy(z_hbm, acc.at[pl.ds(sid * 640, 640)])

    @pl.when(sid == 15)
    def _():
        pltpu.sync_copy(z_hbm.at[pl.ds(0, _ACC_ROWS - 15 * 640)],
                        acc.at[pl.ds(15 * 640, _ACC_ROWS - 15 * 640)])

    plsc.subcore_barrier()

    base = sid * _TILE_E

    def fetch_idx(j, b):
        pltpu.async_copy(src_hbm.at[pl.ds(base + j * _CHUNK, _CHUNK)],
                         si[b], isem[b])
        pltpu.async_copy(dst_hbm.at[pl.ds(base + j * _CHUNK, _CHUNK)],
                         di[b], isem[b])

    def wait_idx(b):
        pltpu.make_async_copy(src_hbm.at[pl.ds(0, _CHUNK)],
                              si[b], isem[b]).wait()
        pltpu.make_async_copy(src_hbm.at[pl.ds(0, _CHUNK)],
                              di[b], isem[b]).wait()

    def gather(b):
        @pl.when(cid == 0)
        def _():
            pltpu.async_copy(hw0_hbm.at[si[b]], rows[b], gsem[b])

        @pl.when(cid == 1)
        def _():
            pltpu.async_copy(hw1_hbm.at[si[b]], rows[b], gsem[b])

    def wait_gather(b):
        @pl.when(cid == 0)
        def _():
            pltpu.make_async_copy(hw0_hbm.at[si[b]], rows[b], gsem[b]).wait()

        @pl.when(cid == 1)
        def _():
            pltpu.make_async_copy(hw1_hbm.at[si[b]], rows[b], gsem[b]).wait()

    fetch_idx(0, 0)
    fetch_idx(1, 1)
    wait_idx(0)
    gather(0)

    def step(i, _):
        for b in range(2):
            j = i * 2 + b
            b1 = (b + 1) % 2

            @pl.when(j + 1 < _NCHUNK)
            def _():
                wait_idx(b1)
                gather(b1)

            wait_gather(b)
            pltpu.sync_copy(rows[b], acc.at[di[b]], add=True)

            @pl.when(j + 2 < _NCHUNK)
            def _():
                fetch_idx(j + 2, b)
        return 0

    lax.fori_loop(0, _NCHUNK // 2, step, 0)
    plsc.subcore_barrier()

    @pl.when(sid < _WT)
    def _():
        for k in range(8):
            nr = 128 if k < 7 else _WROWS - 7 * 128
            r = sid * _WROWS + k * 128
            pltpu.sync_copy(acc.at[pl.ds(r, nr)], r0.at[pl.ds(0, nr)])

            @pl.when(cid == 0)
            def _():
                pltpu.sync_copy(r0.at[pl.ds(0, nr)], s0_hbm.at[pl.ds(r, nr)])

            @pl.when(cid == 1)
            def _():
                pltpu.sync_copy(r0.at[pl.ds(0, nr)], s1_hbm.at[pl.ds(r, nr)])


def _make_deg_kernel(interpret=False):
    return pl.kernel(
        _deg_body,
        out_type=[jax.ShapeDtypeStruct((_N, 16), jnp.float32),
                  jax.ShapeDtypeStruct((_N, 16), jnp.float32)],
        mesh=_sc_mesh,
        scratch_types=[
            pltpu.VMEM((_CHUNK,), jnp.int32),
            pltpu.VMEM((_CHUNK, 16), jnp.float32),
            pltpu.VMEM((_WCH, 16), jnp.float32),
            pltpu.VMEM_SHARED((_ACC_ROWS, 16), jnp.float32),
        ],
        interpret=interpret,
    )


def _make_edge_sum_kernel(interpret=False):
    return pl.kernel(
        _edge_sum_body,
        out_type=[jax.ShapeDtypeStruct((_N, 128), jnp.float32),
                  jax.ShapeDtypeStruct((_N, 128), jnp.float32)],
        mesh=_sc_mesh,
        scratch_types=[
            pltpu.VMEM((_CHUNK,), jnp.int32),
            pltpu.VMEM((_CHUNK,), jnp.int32),
            pltpu.VMEM((_CHUNK,), jnp.int32),
            pltpu.VMEM((_CHUNK,), jnp.int32),
            pltpu.VMEM((_CHUNK, 128), jnp.float32),
            pltpu.VMEM((_CHUNK, 128), jnp.float32),
            pltpu.VMEM_SHARED((_ACC_ROWS, 128), jnp.float32),
            pltpu.SemaphoreType.DMA,
            pltpu.SemaphoreType.DMA,
            pltpu.SemaphoreType.DMA,
            pltpu.SemaphoreType.DMA,
        ],
        interpret=interpret,
    )


_deg_kernel = _make_deg_kernel()
_edge_sum_kernel = _make_edge_sum_kernel()


_BN = 1000


def _k1_body(x_ref, w_ref, p0_ref, p1_ref, hw0_ref, hw1_ref, dis_ref):
    deg = p0_ref[:, :1] + p1_ref[:, :1] + 1.0
    dis = lax.rsqrt(deg)
    hw = jnp.dot(x_ref[...], w_ref[...],
                 preferred_element_type=jnp.float32) * dis
    hw0_ref[...] = hw[:, :128]
    hw1_ref[...] = hw[:, 128:]
    dis_ref[...] = dis


_k1 = pl.pallas_call(
    _k1_body,
    grid=(_N // _BN,),
    in_specs=[
        pl.BlockSpec((_BN, _D), lambda i: (i, 0)),
        pl.BlockSpec((_D, _H), lambda i: (0, 0)),
        pl.BlockSpec((_BN, 16), lambda i: (i, 0)),
        pl.BlockSpec((_BN, 16), lambda i: (i, 0)),
    ],
    out_specs=[
        pl.BlockSpec((_BN, 128), lambda i: (i, 0)),
        pl.BlockSpec((_BN, 128), lambda i: (i, 0)),
        pl.BlockSpec((_BN, 1), lambda i: (i, 0)),
    ],
    out_shape=[jax.ShapeDtypeStruct((_N, 128), jnp.float32),
               jax.ShapeDtypeStruct((_N, 128), jnp.float32),
               jax.ShapeDtypeStruct((_N, 1), jnp.float32)],
)


def _k23_body(s0_ref, s1_ref, hw0_ref, hw1_ref, dis_ref, b_ref, w_ref,
              o0_ref, o1_ref):
    dis = dis_ref[...]
    b = b_ref[...]
    h0 = jnp.maximum(dis * (s0_ref[...] + hw0_ref[...]) + b[:, :128], 0.0)
    h1 = jnp.maximum(dis * (s1_ref[...] + hw1_ref[...]) + b[:, 128:], 0.0)
    h = jnp.concatenate([h0, h1], axis=1)
    hw = jnp.dot(h, w_ref[...], preferred_element_type=jnp.float32) * dis
    o0_ref[...] = hw[:, :128]
    o1_ref[...] = hw[:, 128:]


_k23 = pl.pallas_call(
    _k23_body,
    grid=(_N // _BN,),
    in_specs=[
        pl.BlockSpec((_BN, 128), lambda i: (i, 0)),
        pl.BlockSpec((_BN, 128), lambda i: (i, 0)),
        pl.BlockSpec((_BN, 128), lambda i: (i, 0)),
        pl.BlockSpec((_BN, 128), lambda i: (i, 0)),
        pl.BlockSpec((_BN, 1), lambda i: (i, 0)),
        pl.BlockSpec((1, _H), lambda i: (0, 0)),
        pl.BlockSpec((_H, _H), lambda i: (0, 0)),
    ],
    out_specs=[
        pl.BlockSpec((_BN, 128), lambda i: (i, 0)),
        pl.BlockSpec((_BN, 128), lambda i: (i, 0)),
    ],
    out_shape=[jax.ShapeDtypeStruct((_N, 128), jnp.float32),
               jax.ShapeDtypeStruct((_N, 128), jnp.float32)],
)


def _kobs_body(obs2_ref, lw1_ref, lb1_ref, lw2_ref, lb2_ref, lw3_ref, lb3_ref,
               out_ref):
    obs2 = obs2_ref[...]
    o = jnp.maximum(jnp.dot(obs2, lw1_ref[...],
                            preferred_element_type=jnp.float32)
                    + lb1_ref[...], 0.0)
    o = jnp.maximum(jnp.dot(o, lw2_ref[...],
                            preferred_element_type=jnp.float32)
                    + lb2_ref[...], 0.0)
    o = jnp.dot(o, lw3_ref[...], preferred_element_type=jnp.float32) \
        + lb3_ref[...]
    m = (obs2[:, :1] >= 0.0).astype(jnp.float32)
    gid = lax.broadcasted_iota(jnp.int32, (_G * _L, _G), 0) // _L
    onehot = (gid == lax.broadcasted_iota(jnp.int32, (_G * _L, _G), 1)
              ).astype(jnp.float32)
    rs = lax.dot_general(onehot, m, (((0,), (0,)), ((), ())),
                         preferred_element_type=jnp.float32)
    pooled = lax.dot_general(onehot, o * m, (((0,), (0,)), ((), ())),
                             preferred_element_type=jnp.float32)
    out_ref[...] = pooled / (rs + 1e-9)


_kobs = pl.pallas_call(
    _kobs_body,
    in_specs=[
        pl.BlockSpec((_G * _L, 2), lambda: (0, 0)),
        pl.BlockSpec((2, _H), lambda: (0, 0)),
        pl.BlockSpec((1, _H), lambda: (0, 0)),
        pl.BlockSpec((_H, _H), lambda: (0, 0)),
        pl.BlockSpec((1, _H), lambda: (0, 0)),
        pl.BlockSpec((_H, _H), lambda: (0, 0)),
        pl.BlockSpec((1, _H), lambda: (0, 0)),
    ],
    out_specs=pl.BlockSpec((_G, _H), lambda: (0, 0)),
    out_shape=jax.ShapeDtypeStruct((_G, _H), jnp.float32),
)


def _k4_body(s0_ref, s1_ref, hw0_ref, hw1_ref, dis_ref, b_ref, batch_ref,
             obsp_ref, bw1_ref, bb1_ref, bw2_ref, bb2_ref,
             out_ref, pooled_acc):
    i = pl.program_id(0)
    dis = dis_ref[...]
    b = b_ref[...]
    h0 = dis * (s0_ref[...] + hw0_ref[...]) + b[:, :128]
    h1 = dis * (s1_ref[...] + hw1_ref[...]) + b[:, 128:]
    h = jnp.concatenate([h0, h1], axis=1)
    onehot = (batch_ref[...] ==
              lax.broadcasted_iota(jnp.int32, (_BN, _G), 1)
              ).astype(jnp.float32)
    part = lax.dot_general(onehot, h, (((0,), (0,)), ((), ())),
                           preferred_element_type=jnp.float32)

    @pl.when(i == 0)
    def _():
        pooled_acc[...] = part

    @pl.when(i > 0)
    def _():
        pooled_acc[...] = pooled_acc[...] + part

    @pl.when(i == _N // _BN - 1)
    def _():
        feat = pooled_acc[...] + obsp_ref[...]
        z = jnp.maximum(jnp.dot(feat, bw1_ref[...],
                                preferred_element_type=jnp.float32)
                        + bb1_ref[...], 0.0)
        logits = jnp.dot(z, bw2_ref[...],
                         preferred_element_type=jnp.float32)
        out_ref[...] = 1.0 / (1.0 + jnp.exp(-(logits[:, :1] + bb2_ref[...])))


_k4 = pl.pallas_call(
    _k4_body,
    grid=(_N // _BN,),
    in_specs=[
        pl.BlockSpec((_BN, 128), lambda i: (i, 0)),
        pl.BlockSpec((_BN, 128), lambda i: (i, 0)),
        pl.BlockSpec((_BN, 128), lambda i: (i, 0)),
        pl.BlockSpec((_BN, 128), lambda i: (i, 0)),
        pl.BlockSpec((_BN, 1), lambda i: (i, 0)),
        pl.BlockSpec((1, _H), lambda i: (0, 0)),
        pl.BlockSpec((_BN, 1), lambda i: (i, 0)),
        pl.BlockSpec((_G, _H), lambda i: (0, 0)),
        pl.BlockSpec((_H, _H), lambda i: (0, 0)),
        pl.BlockSpec((1, _H), lambda i: (0, 0)),
        pl.BlockSpec((_H, 128), lambda i: (0, 0)),
        pl.BlockSpec((1, 1), lambda i: (0, 0)),
    ],
    out_specs=pl.BlockSpec((_G, 1), lambda i: (0, 0)),
    out_shape=jax.ShapeDtypeStruct((_G, 1), jnp.float32),
    scratch_shapes=[pltpu.VMEM((_G, _H), jnp.float32)],
)


def kernel(x, edge_index, batch, obs, W1, b1, W2, b2, W3, b3,
           LW1, Lb1, LW2, Lb2, LW3, Lb3, BW1, Bb1, BW2, Bb2):
    pad = _EPAD - _E
    srcp = jnp.concatenate([edge_index[0],
                            jnp.zeros((pad,), edge_index.dtype)])
    dstp = jnp.concatenate([edge_index[1],
                            jnp.full((pad,), _N, edge_index.dtype)])
    z128 = jnp.zeros((640, 128), jnp.float32)
    z8 = jnp.zeros((640, 16), jnp.float32)
    ones8 = jnp.ones((_CHUNK, 16), jnp.float32)

    p0, p1 = _deg_kernel(dstp, z8, ones8)
    hw0, hw1, dis = _k1(x, W1, p0, p1)
    s0, s1 = _edge_sum_kernel(srcp, dstp, hw0, hw1, z128)
    hw0, hw1 = _k23(s0, s1, hw0, hw1, dis, b1.reshape(1, _H), W2)
    s0, s1 = _edge_sum_kernel(srcp, dstp, hw0, hw1, z128)
    hw0, hw1 = _k23(s0, s1, hw0, hw1, dis, b2.reshape(1, _H), W3)
    s0, s1 = _edge_sum_kernel(srcp, dstp, hw0, hw1, z128)

    obsp = _kobs(obs.reshape(_G * _L, 2), LW1, Lb1.reshape(1, _H),
                 LW2, Lb2.reshape(1, _H), LW3, Lb3.reshape(1, _H))

    bw2p = jnp.pad(BW2, ((0, 0), (0, 127)))
    out = _k4(s0, s1, hw0, hw1, dis, b3.reshape(1, _H),
              batch.reshape(_N, 1), obsp, BW1, Bb1.reshape(1, _H),
              bw2p, Bb2.reshape(1, 1))
    return out
